# Initial kernel scaffold; baseline (speedup 1.0000x reference)
import functools

import jax
import jax.numpy as jnp
from jax import lax
from jax.experimental import pallas as pl
from jax.experimental.pallas import tpu as pltpu
from jax.experimental.pallas import tpu_sc as plsc

N = 65536
HIDDEN = 256
NUM_MOL = 2048
TABLE_PAD = 128

NC = 2
NS = 16
NW = NC * NS
CHUNK = N // NW
GROUPS = CHUNK // 16


def _table_body(emb_ref, w1_ref, b1_ref, w2_ref, b2_ref, out_ref):
    h = jnp.dot(emb_ref[...], w1_ref[...], preferred_element_type=jnp.float32)
    h = h + b1_ref[...]
    h = h * jax.nn.sigmoid(h)
    v = jnp.dot(h, w2_ref[...], preferred_element_type=jnp.float32)
    out_ref[...] = v + b2_ref[...]


def _compute_table(emb_p, w1, b1, w2, b2):
    return pl.pallas_call(
        _table_body,
        out_shape=jax.ShapeDtypeStruct((TABLE_PAD, 1), jnp.float32),
    )(emb_p, w1, b1, w2, b2)


def _pool_body(vals_hbm, z_hbm, b_hbm, bn_hbm, out_hbm,
               vals_v, z_v, b_v, bn_v, acc_v):
    wid = lax.axis_index("s") * NC + lax.axis_index("c")
    base = wid * CHUNK
    pltpu.sync_copy(vals_hbm, vals_v)
    pltpu.sync_copy(z_hbm.at[pl.ds(base, CHUNK)], z_v)
    pltpu.sync_copy(b_hbm.at[pl.ds(base, CHUNK)], b_v)
    pltpu.sync_copy(bn_hbm.at[pl.ds(base, CHUNK)], bn_v)

    zeros16 = jnp.zeros((16,), jnp.float32)

    def zero_body(i, carry):
        acc_v[pl.ds(i * 16, 16)] = zeros16
        return carry

    lax.fori_loop(0, NUM_MOL // 16, zero_body, 0)

    lane = lax.iota(jnp.int32, 16)
    is_last = lane == 15

    def body(i, carry):
        off = i * 16
        zi = z_v[pl.ds(off, 16)]
        v = plsc.load_gather(vals_v, [zi])
        b = b_v[pl.ds(off, 16)]
        bn = bn_v[pl.ds(off, 16)]
        c = plsc.cumsum(v)
        diff = b != bn
        plsc.addupdate_scatter(acc_v, [b], c, mask=diff | is_last)
        plsc.addupdate_scatter(acc_v, [bn], -c, mask=diff & (~is_last))
        return carry

    lax.fori_loop(0, GROUPS, body, 0)

    pltpu.sync_copy(acc_v, out_hbm.at[wid])


def _pool_call(vals, z, b, bn):
    mesh = plsc.VectorSubcoreMesh(core_axis_name="c", subcore_axis_name="s")
    f = pl.kernel(
        _pool_body,
        out_type=jax.ShapeDtypeStruct((NW, NUM_MOL), jnp.float32),
        mesh=mesh,
        scratch_types=[
            pltpu.VMEM((TABLE_PAD,), jnp.float32),
            pltpu.VMEM((CHUNK,), jnp.int32),
            pltpu.VMEM((CHUNK,), jnp.int32),
            pltpu.VMEM((CHUNK,), jnp.int32),
            pltpu.VMEM((NUM_MOL,), jnp.float32),
        ],
    )
    return f(vals, z, b, bn)


def _combine_body(p_ref, o_ref):
    o_ref[...] = jnp.sum(p_ref[...], axis=0, keepdims=True)


def _combine(partials):
    return pl.pallas_call(
        _combine_body,
        out_shape=jax.ShapeDtypeStruct((1, NUM_MOL), jnp.float32),
    )(partials)


def kernel(z, pos, batch, emb, W1, b1, W2, b2):
    del pos
    emb_p = jnp.zeros((TABLE_PAD, HIDDEN), jnp.float32).at[: emb.shape[0]].set(emb)
    vals = _compute_table(emb_p, W1, b1.reshape(1, HIDDEN // 2),
                          W2, b2.reshape(1, 1))
    vals = vals.reshape(TABLE_PAD)

    z32 = z.astype(jnp.int32)
    b32 = batch.astype(jnp.int32)
    bn32 = jnp.concatenate([b32[1:], jnp.full((1,), NUM_MOL, jnp.int32)])

    partials = _pool_call(vals, z32, b32, bn32)
    out = _combine(partials)
    return out.reshape(NUM_MOL, 1)


# speedup vs baseline: 11.0862x; 11.0862x over previous
import functools

import jax
import jax.numpy as jnp
from jax import lax
from jax.experimental import pallas as pl
from jax.experimental.pallas import tpu as pltpu
from jax.experimental.pallas import tpu_sc as plsc

N = 65536
HIDDEN = 256
NUM_MOL = 2048
TABLE_PAD = 128

NC = 2
NS = 16
NW = NC * NS
CHUNK = N // NW
GROUPS = CHUNK // 16


def _table_body(emb_ref, w1_ref, b1_ref, w2_ref, b2_ref, out_ref):
    h = jnp.dot(emb_ref[...], w1_ref[...], preferred_element_type=jnp.float32)
    h = h + b1_ref[...]
    h = h * jax.nn.sigmoid(h)
    v = jnp.dot(h, w2_ref[...], preferred_element_type=jnp.float32)
    out_ref[...] = v + b2_ref[...]


def _compute_table(emb_p, w1, b1, w2, b2):
    return pl.pallas_call(
        _table_body,
        out_shape=jax.ShapeDtypeStruct((TABLE_PAD, 1), jnp.float32),
    )(emb_p, w1, b1, w2, b2)


def _pool_body(vals_hbm, z_hbm, b_hbm, bn_hbm, out_hbm,
               vals_v, z_v, b_v, bn_v, acc_v):
    wid = lax.axis_index("s") * NC + lax.axis_index("c")
    base = wid * CHUNK
    pltpu.sync_copy(vals_hbm, vals_v)
    pltpu.sync_copy(z_hbm.at[pl.ds(base, CHUNK)], z_v)
    pltpu.sync_copy(b_hbm.at[pl.ds(base, CHUNK)], b_v)
    pltpu.sync_copy(bn_hbm.at[pl.ds(base, CHUNK)], bn_v)

    zeros16 = jnp.zeros((16,), jnp.float32)

    def zero_body(i, carry):
        acc_v[pl.ds(i * 16, 16)] = zeros16
        return carry

    lax.fori_loop(0, NUM_MOL // 16, zero_body, 0)

    lane = lax.iota(jnp.int32, 16)
    is_last = lane == 15

    def body(i, carry):
        off = i * 16
        zi = z_v[pl.ds(off, 16)]
        v = plsc.load_gather(vals_v, [zi])
        b = b_v[pl.ds(off, 16)]
        bn = bn_v[pl.ds(off, 16)]
        c = plsc.cumsum(v)
        diff = b != bn
        plsc.addupdate_scatter(acc_v, [b], c, mask=diff | is_last)
        plsc.addupdate_scatter(acc_v, [bn], -c, mask=diff & (~is_last))
        return carry

    lax.fori_loop(0, GROUPS, body, 0)

    pltpu.sync_copy(acc_v, out_hbm.at[wid])


def _pool_call(vals, z, b, bn):
    mesh = plsc.VectorSubcoreMesh(core_axis_name="c", subcore_axis_name="s")
    f = pl.kernel(
        _pool_body,
        out_type=jax.ShapeDtypeStruct((NW, NUM_MOL), jnp.float32),
        mesh=mesh,
        compiler_params=pltpu.CompilerParams(needs_layout_passes=False),
        scratch_types=[
            pltpu.VMEM((TABLE_PAD,), jnp.float32),
            pltpu.VMEM((CHUNK,), jnp.int32),
            pltpu.VMEM((CHUNK,), jnp.int32),
            pltpu.VMEM((CHUNK,), jnp.int32),
            pltpu.VMEM((NUM_MOL,), jnp.float32),
        ],
    )
    return f(vals, z, b, bn)


def _combine_body(p_ref, o_ref):
    o_ref[...] = jnp.sum(p_ref[...], axis=0, keepdims=True)


def _combine(partials):
    return pl.pallas_call(
        _combine_body,
        out_shape=jax.ShapeDtypeStruct((1, NUM_MOL), jnp.float32),
    )(partials)


def kernel(z, pos, batch, emb, W1, b1, W2, b2):
    del pos
    emb_p = jnp.zeros((TABLE_PAD, HIDDEN), jnp.float32).at[: emb.shape[0]].set(emb)
    vals = _compute_table(emb_p, W1, b1.reshape(1, HIDDEN // 2),
                          W2, b2.reshape(1, 1))
    vals = vals.reshape(TABLE_PAD)

    z32 = z.astype(jnp.int32)
    b32 = batch.astype(jnp.int32)
    bn32 = jnp.concatenate([b32[1:], jnp.full((1,), NUM_MOL, jnp.int32)])

    partials = _pool_call(vals, z32, b32, bn32)
    out = _combine(partials)
    return out.reshape(NUM_MOL, 1)


# speedup vs baseline: 13.0819x; 1.1800x over previous
import jax
import jax.numpy as jnp
from jax import lax
from jax.experimental import pallas as pl
from jax.experimental.pallas import tpu as pltpu
from jax.experimental.pallas import tpu_sc as plsc

N = 65536
HIDDEN = 256
NUM_MOL = 2048
NUM_Z = 100

NC = 2
NS = 16
NW = NC * NS
CHUNK = N // NW
GROUPS = CHUNK // 16
UNROLL = 4


def _table_body(emb_ref, w1_ref, b1_ref, w2_ref, b2_ref, out_ref):
    h = jnp.dot(emb_ref[...], w1_ref[...], preferred_element_type=jnp.float32)
    h = h + b1_ref[...]
    h = h * jax.nn.sigmoid(h)
    v = jnp.dot(h, w2_ref[...], preferred_element_type=jnp.float32)
    out_ref[...] = v + b2_ref[...]


def _compute_table(emb, w1, b1, w2, b2):
    return pl.pallas_call(
        _table_body,
        out_shape=jax.ShapeDtypeStruct((NUM_Z, 1), jnp.float32),
    )(emb, w1, b1, w2, b2)


def _pool_body(vals_hbm, z_hbm, b_hbm, out_hbm,
               vals_v, z_v, b_v, acc_v, sem):
    wid = lax.axis_index("s") * NC + lax.axis_index("c")
    base = wid * CHUNK
    c1 = pltpu.async_copy(vals_hbm, vals_v, sem)
    c2 = pltpu.async_copy(z_hbm.at[pl.ds(base, CHUNK)], z_v, sem)
    c3 = pltpu.async_copy(b_hbm.at[pl.ds(base, CHUNK)], b_v, sem)

    zeros16 = jnp.zeros((16,), jnp.float32)

    def zero_body(i, carry):
        for u in range(8):
            acc_v[pl.ds((i * 8 + u) * 16, 16)] = zeros16
        return carry

    lax.fori_loop(0, NUM_MOL // 16 // 8, zero_body, 0)

    c1.wait()
    c2.wait()
    c3.wait()

    lane = lax.iota(jnp.int32, 16)
    is_last = lane == 15
    shift_idx = jnp.minimum(lane + 1, 15)

    def group(g):
        off = g * 16
        zi = z_v[pl.ds(off, 16)]
        v = plsc.load_gather(vals_v, [zi])
        b = b_v[pl.ds(off, 16)]
        bs = lax.gather(
            b, shift_idx[:, None],
            lax.GatherDimensionNumbers(offset_dims=(),
                                       collapsed_slice_dims=(0,),
                                       start_index_map=(0,)),
            (1,), mode=lax.GatherScatterMode.PROMISE_IN_BOUNDS)
        c = plsc.cumsum(v)
        diff = (b != bs) & (~is_last)
        plsc.addupdate_scatter(acc_v, [b], c, mask=diff | is_last)
        plsc.addupdate_scatter(acc_v, [bs], -c, mask=diff)

    def body(i, carry):
        for u in range(UNROLL):
            group(i * UNROLL + u)
        return carry

    lax.fori_loop(0, GROUPS // UNROLL, body, 0)

    pltpu.sync_copy(acc_v, out_hbm.at[wid])


def _pool_call(vals, z, b):
    mesh = plsc.VectorSubcoreMesh(core_axis_name="c", subcore_axis_name="s")
    f = pl.kernel(
        _pool_body,
        out_type=jax.ShapeDtypeStruct((NW, NUM_MOL), jnp.float32),
        mesh=mesh,
        compiler_params=pltpu.CompilerParams(needs_layout_passes=False),
        scratch_types=[
            pltpu.VMEM((NUM_Z,), jnp.float32),
            pltpu.VMEM((CHUNK,), jnp.int32),
            pltpu.VMEM((CHUNK,), jnp.int32),
            pltpu.VMEM((NUM_MOL,), jnp.float32),
            pltpu.SemaphoreType.DMA,
        ],
    )
    return f(vals, z, b)


def _combine_body(p_ref, o_ref):
    o_ref[...] = jnp.sum(p_ref[...], axis=0, keepdims=True)


def _combine(partials):
    return pl.pallas_call(
        _combine_body,
        out_shape=jax.ShapeDtypeStruct((1, NUM_MOL), jnp.float32),
    )(partials)


def kernel(z, pos, batch, emb, W1, b1, W2, b2):
    del pos
    vals = _compute_table(emb, W1, b1.reshape(1, HIDDEN // 2),
                          W2, b2.reshape(1, 1))
    vals = vals.reshape(NUM_Z)

    partials = _pool_call(vals, z.astype(jnp.int32), batch.astype(jnp.int32))
    out = _combine(partials)
    return out.reshape(NUM_MOL, 1)


# speedup vs baseline: 14.2502x; 1.0893x over previous
import jax
import jax.numpy as jnp
from jax import lax
from jax.experimental import pallas as pl
from jax.experimental.pallas import tpu as pltpu
from jax.experimental.pallas import tpu_sc as plsc

N = 65536
HIDDEN = 256
NUM_MOL = 2048
NUM_Z = 100

NC = 2
NS = 16
NW = NC * NS
CHUNK = N // NW
GROUPS = CHUNK // 16
UNROLL = 4


def _table_body(emb_ref, w1_ref, b1_ref, w2_ref, b2_ref, out_ref):
    h = jnp.dot(emb_ref[...], w1_ref[...], preferred_element_type=jnp.float32)
    h = h + b1_ref[...]
    h = h * jax.nn.sigmoid(h)
    v = jnp.sum(h * w2_ref[...], axis=1)
    out_ref[...] = v + b2_ref[...]


def _compute_table(emb, w1, b1, w2, b2):
    return pl.pallas_call(
        _table_body,
        out_shape=jax.ShapeDtypeStruct((NUM_Z,), jnp.float32),
    )(emb, w1, b1, w2, b2)


def _pool_body(vals_hbm, z_hbm, b_hbm, out_hbm,
               vals_v, z_v, b_v, acc_v, sem):
    wid = lax.axis_index("s") * NC + lax.axis_index("c")
    base = wid * CHUNK
    c1 = pltpu.async_copy(vals_hbm, vals_v, sem)
    c2 = pltpu.async_copy(z_hbm.at[pl.ds(base, CHUNK)], z_v, sem)
    c3 = pltpu.async_copy(b_hbm.at[pl.ds(base, CHUNK)], b_v, sem)

    zeros16 = jnp.zeros((16,), jnp.float32)

    def zero_body(i, carry):
        for u in range(8):
            acc_v[pl.ds((i * 8 + u) * 16, 16)] = zeros16
        return carry

    lax.fori_loop(0, NUM_MOL // 16 // 8, zero_body, 0)

    c1.wait()
    c2.wait()
    c3.wait()

    lane = lax.iota(jnp.int32, 16)
    is_last = lane == 15
    shift_idx = jnp.minimum(lane + 1, 15)

    def group(g):
        off = g * 16
        zi = z_v[pl.ds(off, 16)]
        v = plsc.load_gather(vals_v, [zi])
        b = b_v[pl.ds(off, 16)]
        bs = lax.gather(
            b, shift_idx[:, None],
            lax.GatherDimensionNumbers(offset_dims=(),
                                       collapsed_slice_dims=(0,),
                                       start_index_map=(0,)),
            (1,), mode=lax.GatherScatterMode.PROMISE_IN_BOUNDS)
        c = plsc.cumsum(v)
        diff = (b != bs) & (~is_last)
        plsc.addupdate_scatter(acc_v, [b], c, mask=diff | is_last)
        plsc.addupdate_scatter(acc_v, [bs], -c, mask=diff)

    def body(i, carry):
        for u in range(UNROLL):
            group(i * UNROLL + u)
        return carry

    lax.fori_loop(0, GROUPS // UNROLL, body, 0)

    pltpu.sync_copy(acc_v, out_hbm.at[wid])


def _pool_call(vals, z, b):
    mesh = plsc.VectorSubcoreMesh(core_axis_name="c", subcore_axis_name="s")
    f = pl.kernel(
        _pool_body,
        out_type=jax.ShapeDtypeStruct((NW, NUM_MOL), jnp.float32),
        mesh=mesh,
        compiler_params=pltpu.CompilerParams(needs_layout_passes=False),
        scratch_types=[
            pltpu.VMEM((NUM_Z,), jnp.float32),
            pltpu.VMEM((CHUNK,), jnp.int32),
            pltpu.VMEM((CHUNK,), jnp.int32),
            pltpu.VMEM((NUM_MOL,), jnp.float32),
            pltpu.SemaphoreType.DMA,
        ],
    )
    return f(vals, z, b)


def _combine_body(p_ref, o_ref):
    o_ref[...] = jnp.sum(p_ref[...], axis=0, keepdims=True)


def _combine(partials):
    return pl.pallas_call(
        _combine_body,
        out_shape=jax.ShapeDtypeStruct((1, NUM_MOL), jnp.float32),
    )(partials)


def kernel(z, pos, batch, emb, W1, b1, W2, b2):
    del pos
    vals = _compute_table(emb, W1, b1.reshape(1, HIDDEN // 2),
                          W2.reshape(1, HIDDEN // 2), b2)

    partials = _pool_call(vals, z.astype(jnp.int32), batch.astype(jnp.int32))
    out = _combine(partials)
    return out.reshape(NUM_MOL, 1)


# speedup vs baseline: 14.3326x; 1.0058x over previous
import jax
import jax.numpy as jnp
from jax import lax
from jax.experimental import pallas as pl
from jax.experimental.pallas import tpu as pltpu
from jax.experimental.pallas import tpu_sc as plsc

N = 65536
HIDDEN = 256
NUM_MOL = 2048
NUM_Z = 100

NC = 2
NS = 16
NW = NC * NS
CHUNK = N // NW
GROUPS = CHUNK // 16
UNROLL = 1


def _table_body(emb_ref, w1_ref, b1_ref, w2_ref, b2_ref, out_ref):
    h = jnp.dot(emb_ref[...], w1_ref[...], preferred_element_type=jnp.float32)
    h = h + b1_ref[...]
    h = h * jax.nn.sigmoid(h)
    v = jnp.sum(h * w2_ref[...], axis=1)
    out_ref[...] = v + b2_ref[...]


def _compute_table(emb, w1, b1, w2, b2):
    return pl.pallas_call(
        _table_body,
        out_shape=jax.ShapeDtypeStruct((NUM_Z,), jnp.float32),
    )(emb, w1, b1, w2, b2)


def _pool_body(vals_hbm, z_hbm, b_hbm, out_hbm,
               vals_v, z_v, b_v, acc_v, sem):
    wid = lax.axis_index("s") * NC + lax.axis_index("c")
    base = wid * CHUNK
    c1 = pltpu.async_copy(vals_hbm, vals_v, sem)
    c2 = pltpu.async_copy(z_hbm.at[pl.ds(base, CHUNK)], z_v, sem)
    c3 = pltpu.async_copy(b_hbm.at[pl.ds(base, CHUNK)], b_v, sem)

    zeros16 = jnp.zeros((16,), jnp.float32)

    def zero_body(i, carry):
        for u in range(8):
            acc_v[pl.ds((i * 8 + u) * 16, 16)] = zeros16
        return carry

    lax.fori_loop(0, NUM_MOL // 16 // 8, zero_body, 0)

    c1.wait()
    c2.wait()
    c3.wait()

    lane = lax.iota(jnp.int32, 16)
    is_last = lane == 15
    shift_idx = jnp.minimum(lane + 1, 15)

    def group(g):
        off = g * 16
        zi = z_v[pl.ds(off, 16)]
        v = plsc.load_gather(vals_v, [zi])
        b = b_v[pl.ds(off, 16)]
        bs = lax.gather(
            b, shift_idx[:, None],
            lax.GatherDimensionNumbers(offset_dims=(),
                                       collapsed_slice_dims=(0,),
                                       start_index_map=(0,)),
            (1,), mode=lax.GatherScatterMode.PROMISE_IN_BOUNDS)
        c = plsc.cumsum(v)
        diff = (b != bs) & (~is_last)
        plsc.addupdate_scatter(acc_v, [b], c, mask=diff | is_last)
        plsc.addupdate_scatter(acc_v, [bs], -c, mask=diff)

    def body(i, carry):
        for u in range(UNROLL):
            group(i * UNROLL + u)
        return carry

    lax.fori_loop(0, GROUPS // UNROLL, body, 0)

    pltpu.sync_copy(acc_v, out_hbm.at[wid])


def _pool_call(vals, z, b):
    mesh = plsc.VectorSubcoreMesh(core_axis_name="c", subcore_axis_name="s")
    f = pl.kernel(
        _pool_body,
        out_type=jax.ShapeDtypeStruct((NW, NUM_MOL), jnp.float32),
        mesh=mesh,
        compiler_params=pltpu.CompilerParams(needs_layout_passes=False),
        scratch_types=[
            pltpu.VMEM((NUM_Z,), jnp.float32),
            pltpu.VMEM((CHUNK,), jnp.int32),
            pltpu.VMEM((CHUNK,), jnp.int32),
            pltpu.VMEM((NUM_MOL,), jnp.float32),
            pltpu.SemaphoreType.DMA,
        ],
    )
    return f(vals, z, b)


def _combine_body(p_ref, o_ref):
    o_ref[...] = jnp.sum(p_ref[...], axis=0, keepdims=True)


def _combine(partials):
    return pl.pallas_call(
        _combine_body,
        out_shape=jax.ShapeDtypeStruct((1, NUM_MOL), jnp.float32),
    )(partials)


def kernel(z, pos, batch, emb, W1, b1, W2, b2):
    del pos
    vals = _compute_table(emb, W1, b1.reshape(1, HIDDEN // 2),
                          W2.reshape(1, HIDDEN // 2), b2)

    partials = _pool_call(vals, z.astype(jnp.int32), batch.astype(jnp.int32))
    out = _combine(partials)
    return out.reshape(NUM_MOL, 1)
